# baseline (speedup 1.0000x reference)
import jax
import jax.numpy as jnp
import numpy as np
from jax.experimental import pallas as pl
from jax.experimental.pallas import tpu as pltpu

D = 256
G = D // 4
K = 64
BLK = 512
KHI = 53


def _qmul(a, b):
    aw, ax, ay, az = a[..., 0], a[..., 1], a[..., 2], a[..., 3]
    bw, bx, by, bz = b[..., 0], b[..., 1], b[..., 2], b[..., 3]
    rw = aw * bw - ax * bx - ay * by - az * bz
    rx = aw * bx + ax * bw + ay * bz - az * by
    ry = aw * by - ax * bz + ay * bw + az * bx
    rz = aw * bz + ax * by - ay * bx + az * bw
    return jnp.stack([rw, rx, ry, rz], axis=-1)


def _qconj(q):
    return q * jnp.array([1.0, -1.0, -1.0, -1.0], dtype=q.dtype)


def _rot_coeffs(q_L, q_R):
    eye = jnp.eye(4, dtype=jnp.float32)
    col = _qmul(_qmul(q_L[:, None, :], eye[None, :, :]), _qconj(q_R)[:, None, :])
    sel = np.zeros((7, 4, 4), np.float32)
    for o in range(-3, 4):
        for i in range(4):
            if 0 <= i + o < 4:
                sel[o + 3, i, i + o] = 1.0
    sel = jnp.asarray(sel)
    fwd = jnp.einsum('gji,oij->ogi', col, sel,
                     precision=jax.lax.Precision.HIGHEST).reshape(7, D)
    eye_g = jnp.eye(G, dtype=jnp.float32)
    M = jnp.swapaxes(col, 1, 2)
    Bi = (eye_g[:, None, :, None] * M[:, :, None, :]).reshape(D, D)
    return fwd, Bi.astype(jnp.bfloat16)


def _shift(a, o):
    if o == 0:
        return a
    if o > 0:
        return jnp.concatenate([a[:, o:], a[:, :o]], axis=1)
    m = -o
    return jnp.concatenate([a[:, D + o:], a[:, :D + o]], axis=1)


def _body(x_ref, band_ref, bi_ref, cen_ref, xhat_ref, idx_ref):
    xb = x_ref[...]
    s = jnp.sum(xb * xb, axis=1, keepdims=True)
    norm = jnp.maximum(jnp.sqrt(s), 1e-8)
    u = xb / norm

    v = jnp.zeros_like(u)
    for o in range(-3, 4):
        v = v + band_ref[o + 3, :][None, :] * _shift(u, o)

    ua = jnp.abs(v)
    nb = KHI - (K // 2 - 1)

    def bnd(t):
        if t >= nb:
            return 3.0
        k = K // 2 - 1 + t
        return (cen_ref[k] + cen_ref[k + 1]) * 0.5

    def cv(c):
        return cen_ref[K // 2 - 1 + min(c, nb)]

    def sel(ms, f, base=0):
        if not ms:
            return f(base)
        if base >= nb:
            return f(base)
        m, rest = ms[0], ms[1:]
        return jnp.where(m, sel(rest, f, base + (1 << len(rest))),
                         sel(rest, f, base))

    ms = []
    for lev in range(5):
        shift = 5 - lev
        t = sel(ms, lambda b: bnd((b << shift) + (1 << (shift - 1)) - 1))
        ms.append(ua > t)

    cf = jnp.zeros(v.shape, jnp.float32)
    for i, m in enumerate(ms):
        cf = cf + jnp.where(m, float(1 << (4 - i)), 0.0)
    ci = cf.astype(jnp.int32)
    vpos = sel(ms, cv)
    pos = v > 0.0
    idx = jnp.where(pos, (K // 2 - 1) + ci, K // 2 - ci)
    val = jnp.where(v >= 0.0, vpos, -vpos)

    w = jax.lax.dot_general(
        val.astype(jnp.bfloat16), bi_ref[...], (((1,), (0,)), ((), ())),
        preferred_element_type=jnp.float32)

    xhat_ref[...] = w * norm
    idx_ref[...] = idx


def kernel(x, q_L, q_R, centroids):
    n = x.shape[0]
    band, bi = _rot_coeffs(q_L, q_R)
    xhat, idx = pl.pallas_call(
        _body,
        grid=(n // BLK,),
        in_specs=[
            pl.BlockSpec((BLK, D), lambda i: (i, 0)),
            pl.BlockSpec((7, D), lambda i: (0, 0)),
            pl.BlockSpec((D, D), lambda i: (0, 0)),
            pl.BlockSpec(memory_space=pltpu.SMEM),
        ],
        out_specs=[
            pl.BlockSpec((BLK, D), lambda i: (i, 0)),
            pl.BlockSpec((BLK, D), lambda i: (i, 0)),
        ],
        out_shape=[
            jax.ShapeDtypeStruct((n, D), jnp.float32),
            jax.ShapeDtypeStruct((n, D), jnp.int32),
        ],
    )(x, band, bi, centroids)
    return xhat, idx


# speedup vs baseline: 1.0250x; 1.0250x over previous
import jax
import jax.numpy as jnp
import numpy as np
from jax.experimental import pallas as pl
from jax.experimental.pallas import tpu as pltpu

D = 256
G = D // 4
K = 64
BLK = 512
KHI = 53


def _qmul(a, b):
    aw, ax, ay, az = a[..., 0], a[..., 1], a[..., 2], a[..., 3]
    bw, bx, by, bz = b[..., 0], b[..., 1], b[..., 2], b[..., 3]
    rw = aw * bw - ax * bx - ay * by - az * bz
    rx = aw * bx + ax * bw + ay * bz - az * by
    ry = aw * by - ax * bz + ay * bw + az * bx
    rz = aw * bz + ax * by - ay * bx + az * bw
    return jnp.stack([rw, rx, ry, rz], axis=-1)


def _qconj(q):
    return q * jnp.array([1.0, -1.0, -1.0, -1.0], dtype=q.dtype)


def _rot_coeffs(q_L, q_R):
    eye = jnp.eye(4, dtype=jnp.float32)
    col = _qmul(_qmul(q_L[:, None, :], eye[None, :, :]), _qconj(q_R)[:, None, :])
    sel = np.zeros((7, 4, 4), np.float32)
    for o in range(-3, 4):
        for i in range(4):
            if 0 <= i + o < 4:
                sel[o + 3, i, i + o] = 1.0
    sel = jnp.asarray(sel)
    fwd = jnp.einsum('gji,oij->ogi', col, sel,
                     precision=jax.lax.Precision.HIGHEST).reshape(7, D)
    inv = jnp.einsum('gij,oij->ogi', col, sel,
                     precision=jax.lax.Precision.HIGHEST).reshape(7, D)
    return jnp.concatenate([fwd, inv])


def _shift(a, o):
    if o == 0:
        return a
    if o > 0:
        return jnp.concatenate([a[:, o:], a[:, :o]], axis=1)
    m = -o
    return jnp.concatenate([a[:, D + o:], a[:, :D + o]], axis=1)


def _body(x_ref, band_ref, cen_ref, xhat_ref, idx_ref):
    xb = x_ref[...]
    s = jnp.sum(xb * xb, axis=1, keepdims=True)
    norm = jnp.maximum(jnp.sqrt(s), 1e-8)
    u = xb / norm

    v = jnp.zeros_like(u)
    for o in range(-3, 4):
        v = v + band_ref[o + 3, :][None, :] * _shift(u, o)

    ua = jnp.abs(v)
    nb = KHI - (K // 2 - 1)

    def bnd(t):
        if t >= nb:
            return 3.0
        k = K // 2 - 1 + t
        return (cen_ref[k] + cen_ref[k + 1]) * 0.5

    def cv(c):
        return cen_ref[K // 2 - 1 + min(c, nb)]

    def sel(ms, f, base=0):
        if not ms:
            return f(base)
        if base >= nb:
            return f(base)
        m, rest = ms[0], ms[1:]
        return jnp.where(m, sel(rest, f, base + (1 << len(rest))),
                         sel(rest, f, base))

    ms = []
    for lev in range(5):
        shift = 5 - lev
        t = sel(ms, lambda b: bnd((b << shift) + (1 << (shift - 1)) - 1))
        ms.append(ua > t)

    cf = jnp.zeros(v.shape, jnp.float32)
    for i, m in enumerate(ms):
        cf = cf + jnp.where(m, float(1 << (4 - i)), 0.0)
    ci = cf.astype(jnp.int32)
    vpos = sel(ms, cv)
    pos = v > 0.0
    idx = jnp.where(pos, (K // 2 - 1) + ci, K // 2 - ci)
    val = jnp.where(v >= 0.0, vpos, -vpos)

    w = jnp.zeros_like(val)
    for o in range(-3, 4):
        w = w + band_ref[o + 10, :][None, :] * _shift(val, o)

    xhat_ref[...] = w * norm
    idx_ref[...] = idx


def kernel(x, q_L, q_R, centroids):
    n = x.shape[0]
    band = _rot_coeffs(q_L, q_R)
    xhat, idx = pl.pallas_call(
        _body,
        grid=(n // BLK,),
        in_specs=[
            pl.BlockSpec((BLK, D), lambda i: (i, 0)),
            pl.BlockSpec((14, D), lambda i: (0, 0)),
            pl.BlockSpec(memory_space=pltpu.SMEM),
        ],
        out_specs=[
            pl.BlockSpec((BLK, D), lambda i: (i, 0)),
            pl.BlockSpec((BLK, D), lambda i: (i, 0)),
        ],
        out_shape=[
            jax.ShapeDtypeStruct((n, D), jnp.float32),
            jax.ShapeDtypeStruct((n, D), jnp.int32),
        ],
    )(x, band, centroids)
    return xhat, idx
